# baseline (speedup 1.0000x reference)
import functools

import jax
import jax.numpy as jnp
from jax import lax
from jax.experimental import pallas as pl
from jax.experimental.pallas import tpu as pltpu

_SCORE_THR = 0.05
_MAX_OUT = 300
_LANES = 128
_S = 4
_KP = 2


def _tree(op, sw, nr):
    cm = sw[0:8]
    for k in range(1, nr // 8):
        cm = op(cm, sw[8 * k:8 * k + 8])
    return cm


def _red2(red, m8):
    return red(red(m8, axis=0, keepdims=True), axis=1, keepdims=True)


def _nms_kernel(b_ref, s_ref, out_ref, sw_ref, area_ref, *, rows):
    x1 = b_ref[0]
    y1 = b_ref[1]
    x2 = b_ref[2]
    y2 = b_ref[3]
    area_ref[...] = (x2 - x1) * (y2 - y1)
    s = s_ref[...]
    sw_ref[...] = jnp.where(s > _SCORE_THR, s, 0.0)
    out_ref[...] = jnp.zeros_like(out_ref)
    lin = (lax.broadcasted_iota(jnp.int32, (rows, _LANES), 0) * _LANES
           + lax.broadcasted_iota(jnp.int32, (rows, _LANES), 1))
    lane = lax.broadcasted_iota(jnp.int32, (1, _LANES), 1)
    subi = lax.broadcasted_iota(jnp.int32, (8, 1), 0)
    n = _S * _KP
    pr = rows // _S

    def cond(c):
        cnt, alive = c
        return (cnt < _MAX_OUT) & (alive > 0)

    def body(c):
        cnt, alive = c
        sw = sw_ref[...]
        area = area_ref[...]

        swm = [sw[p * pr:(p + 1) * pr] for p in range(_S)]
        plist = [[] for _ in range(_S)]
        for _ in range(_KP):
            for p in range(_S):
                sl = slice(p * pr, (p + 1) * pr)
                g = _red2(jnp.max, _tree(jnp.maximum, swm[p], pr))
                eq = swm[p] == g
                idx = _red2(jnp.min, _tree(jnp.minimum,
                                           jnp.where(eq, lin[sl],
                                                     jnp.int32(2**30)), pr))
                sel = lin[sl] == idx
                cs = [_red2(jnp.sum, _tree(jnp.add, jnp.where(sel, a[sl], 0.0),
                                           pr))
                      for a in (x1, y1, x2, y2)]
                swm[p] = jnp.where(sel, 0.0, swm[p])
                plist[p].append((g, idx, cs[0], cs[1], cs[2], cs[3],
                                 g > _SCORE_THR))
        cand = [t for pl_ in plist for t in pl_]

        pred = {}
        for i in range(n):
            for j in range(i + 1, n):
                w = ((cand[i][0] > cand[j][0])
                     | ((cand[i][0] == cand[j][0])
                        & (cand[i][1] < cand[j][1])))
                pred[(i, j)] = w
                pred[(j, i)] = ~w

        base = []
        for i in range(n):
            b = cand[i][6]
            for p in range(_S):
                if p == i // _KP:
                    continue
                last = p * _KP + _KP - 1
                b = b & (pred[(i, last)] | ~cand[last][6])
            base.append(b)

        areas = [(t[4] - t[2]) * (t[5] - t[3]) for t in cand]
        kmask = {}
        for i in range(n):
            for j in range(i + 1, n):
                iw = jnp.maximum(jnp.minimum(cand[i][4], cand[j][4])
                                 - jnp.maximum(cand[i][2], cand[j][2]), 0.0)
                ih = jnp.maximum(jnp.minimum(cand[i][5], cand[j][5])
                                 - jnp.maximum(cand[i][3], cand[j][3]), 0.0)
                ov = 3.0 * (iw * ih) >= areas[i] + areas[j] + 1e-6
                kmask[(i, j)] = ov
                kmask[(j, i)] = ov
        acc = list(base)
        for _ in range(n):
            acc = [base[i]
                   & ~functools.reduce(
                       jnp.logical_or,
                       [pred[(d, i)] & acc[d] & kmask[(d, i)]
                        for d in range(n) if d != i])
                   for i in range(n)]

        keep = None
        for i in range(n):
            iw = jnp.maximum(jnp.minimum(cand[i][4], x2)
                             - jnp.maximum(cand[i][2], x1), 0.0)
            ih = jnp.maximum(jnp.minimum(cand[i][5], y2)
                             - jnp.maximum(cand[i][3], y1), 0.0)
            kill = acc[i] & (3.0 * (iw * ih) >= areas[i] + area + 1e-6)
            keep = ~kill if keep is None else keep & ~kill
        sw_ref[...] = jnp.where(keep, sw, 0.0)

        blk = jnp.zeros((8, _LANES), jnp.float32)
        tot = jnp.zeros((1, 1), jnp.int32)
        for i in range(n):
            o = functools.reduce(
                jnp.add,
                [jnp.where(acc[d] & pred[(d, i)], 1, 0)
                 for d in range(n) if d != i])
            det = (jnp.where(lane == 0, cand[i][2], 0.0)
                   + jnp.where(lane == 1, cand[i][3], 0.0)
                   + jnp.where(lane == 2, cand[i][4], 0.0)
                   + jnp.where(lane == 3, cand[i][5], 0.0)
                   + jnp.where(lane == 4, cand[i][0], 0.0))
            blk = blk + jnp.where((subi == o) & acc[i], det, 0.0)
            tot = tot + jnp.where(acc[i], 1, 0)
        out_ref[pl.ds(cnt, 8), :] = blk

        code = jnp.sum(tot)
        return cnt + code, code

    lax.while_loop(cond, body, (jnp.int32(0), jnp.int32(1)))


def kernel(boxes, scores):
    nb = boxes.shape[0]
    rows = -(-nb // _LANES)
    rows = -(-rows // (8 * _S)) * (8 * _S)
    p = rows * _LANES
    b4 = jnp.pad(boxes.T, ((0, 0), (0, p - nb))).reshape(4, rows, _LANES)
    sp = jnp.pad(scores, (0, p - nb)).reshape(rows, _LANES)
    out = pl.pallas_call(
        functools.partial(_nms_kernel, rows=rows),
        out_shape=jax.ShapeDtypeStruct((_MAX_OUT + 12, _LANES), jnp.float32),
        scratch_shapes=[pltpu.VMEM((rows, _LANES), jnp.float32),
                        pltpu.VMEM((rows, _LANES), jnp.float32)],
    )(b4, sp)
    return out[:_MAX_OUT, :5]
